# baseline (speedup 1.0000x reference)
import functools

import jax
import jax.numpy as jnp
from jax.experimental import pallas as pl
from jax.experimental.pallas import tpu as pltpu

NEG_INF = -1000000000.0
_BR = 256
_BC = 384


def _softmax3_block(x_ref, o_ref, *, n, block_rows):
    w = 3 * n
    i = pl.program_id(0)
    j = pl.program_id(1)
    last_i = pl.num_programs(0) - 1
    last_j = pl.num_programs(1) - 1
    x = x_ref[...]
    lane = jax.lax.broadcasted_iota(jnp.int32, x.shape, 1)
    mod3 = lane % 3

    last_col_lane = (w - 2) - last_j * _BC

    def _mask_col():
        xx = jnp.where(lane >= last_col_lane - 1 + 3, NEG_INF, x)
        return jnp.where(lane == last_col_lane, NEG_INF, xx)

    x = jax.lax.cond(j == last_j, _mask_col, lambda: x)

    row = jax.lax.broadcasted_iota(jnp.int32, x.shape, 0) + i * block_rows

    def _mask_row():
        return jnp.where((row == n - 1) & (mod3 == 0), NEG_INF, x)

    x = jax.lax.cond(i == last_i, _mask_row, lambda: x)

    u = pltpu.roll(x, _BC - 1, 1)
    v = pltpu.roll(x, _BC - 2, 1)
    p = pltpu.roll(x, 1, 1)
    q = pltpu.roll(x, 2, 1)
    o1 = jnp.where(mod3 == 0, u, jnp.where(mod3 == 1, p, q))
    o2 = jnp.where(mod3 == 0, v, jnp.where(mod3 == 1, u, p))
    o_ref[...] = 1.0 / (1.0 + jnp.exp(o1 - x) + jnp.exp(o2 - x))


def kernel(log_edge_flows):
    n = log_edge_flows.shape[0]
    w = 3 * n
    x2d = log_edge_flows.reshape(n, w)
    grid = (pl.cdiv(n, _BR), pl.cdiv(w, _BC))
    out = pl.pallas_call(
        functools.partial(_softmax3_block, n=n, block_rows=_BR),
        grid=grid,
        in_specs=[pl.BlockSpec((_BR, _BC), lambda i, j: (i, j))],
        out_specs=pl.BlockSpec((_BR, _BC), lambda i, j: (i, j)),
        out_shape=jax.ShapeDtypeStruct((n, w), jnp.float32),
    )(x2d)
    return out.reshape(n, n, 3)


# speedup vs baseline: 29.6588x; 29.6588x over previous
import functools

import jax
import jax.numpy as jnp
from jax.experimental import pallas as pl

NEG_INF = -1000000000.0
_BR = 128


def _softmax3_block(x_ref, o_ref, *, n, block_rows):
    i = pl.program_id(0)
    a0 = x_ref[0]
    a1 = x_ref[1]
    a2 = x_ref[2]
    row = jax.lax.broadcasted_iota(jnp.int32, a0.shape, 0) + i * block_rows
    col = jax.lax.broadcasted_iota(jnp.int32, a0.shape, 1)
    a0 = jnp.where(row == n - 1, NEG_INF, a0)
    a1 = jnp.where(col == n - 1, NEG_INF, a1)
    m = jnp.maximum(jnp.maximum(a0, a1), a2)
    e0 = jnp.exp(a0 - m)
    e1 = jnp.exp(a1 - m)
    e2 = jnp.exp(a2 - m)
    inv = 1.0 / (e0 + e1 + e2)
    o_ref[0] = e0 * inv
    o_ref[1] = e1 * inv
    o_ref[2] = e2 * inv


def kernel(log_edge_flows):
    n = log_edge_flows.shape[0]
    x = jnp.transpose(log_edge_flows, (2, 0, 1))
    grid = (pl.cdiv(n, _BR),)
    out = pl.pallas_call(
        functools.partial(_softmax3_block, n=n, block_rows=_BR),
        grid=grid,
        in_specs=[pl.BlockSpec((3, _BR, n), lambda i: (0, i, 0))],
        out_specs=pl.BlockSpec((3, _BR, n), lambda i: (0, i, 0)),
        out_shape=jax.ShapeDtypeStruct((3, n, n), jnp.float32),
    )(x)
    return jnp.transpose(out, (1, 2, 0))


# speedup vs baseline: 30.0000x; 1.0115x over previous
import functools

import jax
import jax.numpy as jnp
from jax.experimental import pallas as pl

NEG_INF = -1000000000.0
_BR = 192


def _softmax3_block(x_ref, o_ref, *, n, block_rows):
    i = pl.program_id(0)
    a0 = x_ref[0]
    a1 = x_ref[1]
    a2 = x_ref[2]
    row = jax.lax.broadcasted_iota(jnp.int32, a0.shape, 0) + i * block_rows
    col = jax.lax.broadcasted_iota(jnp.int32, a0.shape, 1)
    a0 = jnp.where(row == n - 1, NEG_INF, a0)
    a1 = jnp.where(col == n - 1, NEG_INF, a1)
    m = jnp.maximum(jnp.maximum(a0, a1), a2)
    e0 = jnp.exp(a0 - m)
    e1 = jnp.exp(a1 - m)
    e2 = jnp.exp(a2 - m)
    inv = 1.0 / (e0 + e1 + e2)
    o_ref[0] = e0 * inv
    o_ref[1] = e1 * inv
    o_ref[2] = e2 * inv


def kernel(log_edge_flows):
    n = log_edge_flows.shape[0]
    x = jnp.transpose(log_edge_flows, (2, 0, 1))
    grid = (pl.cdiv(n, _BR),)
    out = pl.pallas_call(
        functools.partial(_softmax3_block, n=n, block_rows=_BR),
        grid=grid,
        in_specs=[pl.BlockSpec((3, _BR, n), lambda i: (0, i, 0))],
        out_specs=pl.BlockSpec((3, _BR, n), lambda i: (0, i, 0)),
        out_shape=jax.ShapeDtypeStruct((3, n, n), jnp.float32),
    )(x)
    return jnp.transpose(out, (1, 2, 0))
